# baseline (speedup 1.0000x reference)
import jax
import jax.numpy as jnp
from jax.experimental import pallas as pl
from jax.experimental.pallas import tpu as pltpu


_SHAPE = (8, 4, 2, 262144)
_L = _SHAPE[-1]
_NROWS = 64
_S = 56
_K = 28


def _copy_kernel(in_ref, out_ref, *scratch):
    bufs = scratch[:_S]
    rsems = scratch[_S:2 * _S]
    wsems = scratch[2 * _S:3 * _S]

    def row_idx(r):
        return r >> 3, (r >> 1) & 3, r & 1

    def read(i):
        s = i % _S
        b, st, ch = row_idx(i)
        return pltpu.make_async_copy(in_ref.at[b, st, ch], bufs[s], rsems[s])

    def write(i):
        s = i % _S
        b, st, ch = row_idx(i)
        return pltpu.make_async_copy(bufs[s], out_ref.at[b, st, ch], wsems[s])

    waited = set()
    for k in range(_K):
        read(k).start()
    for i in range(_NROWS):
        j = i + _K
        if j < _NROWS:
            if j >= _S:
                write(j - _S).wait()
                waited.add(j - _S)
            read(j).start()
        read(i).wait()
        write(i).start()
    for i in range(_NROWS):
        if i not in waited:
            write(i).wait()


def kernel(stems):
    return pl.pallas_call(
        _copy_kernel,
        out_shape=jax.ShapeDtypeStruct(_SHAPE, jnp.float32),
        in_specs=[pl.BlockSpec(memory_space=pltpu.MemorySpace.HBM)],
        out_specs=pl.BlockSpec(memory_space=pltpu.MemorySpace.HBM),
        compiler_params=pltpu.CompilerParams(vmem_limit_bytes=100 * 1024 * 1024),
        scratch_shapes=(
            [pltpu.VMEM((_L,), jnp.float32)] * _S
            + [pltpu.SemaphoreType.DMA] * (2 * _S)
        ),
    )(stems)


# speedup vs baseline: 1.0010x; 1.0010x over previous
import jax
import jax.numpy as jnp
from jax.experimental import pallas as pl
from jax.experimental.pallas import tpu as pltpu


_SHAPE = (8, 4, 2, 262144)
_L = _SHAPE[-1]
_NROWS = 64
_S = 56
_K = 48


def _copy_kernel(in_ref, out_ref, *scratch):
    bufs = scratch[:_S]
    rsems = scratch[_S:2 * _S]
    wsems = scratch[2 * _S:3 * _S]

    def row_idx(r):
        return r >> 3, (r >> 1) & 3, r & 1

    def read(i):
        s = i % _S
        b, st, ch = row_idx(i)
        return pltpu.make_async_copy(in_ref.at[b, st, ch], bufs[s], rsems[s])

    def write(i):
        s = i % _S
        b, st, ch = row_idx(i)
        return pltpu.make_async_copy(bufs[s], out_ref.at[b, st, ch], wsems[s])

    waited = set()
    for k in range(_K):
        read(k).start()
    for i in range(_NROWS):
        j = i + _K
        if j < _NROWS:
            if j >= _S:
                write(j - _S).wait()
                waited.add(j - _S)
            read(j).start()
        read(i).wait()
        write(i).start()
    for i in range(_NROWS):
        if i not in waited:
            write(i).wait()


def kernel(stems):
    return pl.pallas_call(
        _copy_kernel,
        out_shape=jax.ShapeDtypeStruct(_SHAPE, jnp.float32),
        in_specs=[pl.BlockSpec(memory_space=pltpu.MemorySpace.HBM)],
        out_specs=pl.BlockSpec(memory_space=pltpu.MemorySpace.HBM),
        compiler_params=pltpu.CompilerParams(vmem_limit_bytes=100 * 1024 * 1024),
        scratch_shapes=(
            [pltpu.VMEM((_L,), jnp.float32)] * _S
            + [pltpu.SemaphoreType.DMA] * (2 * _S)
        ),
    )(stems)


# speedup vs baseline: 1.0446x; 1.0435x over previous
import jax
import jax.numpy as jnp
from jax.experimental import pallas as pl
from jax.experimental.pallas import tpu as pltpu


_SHAPE = (8, 4, 2, 262144)
_L = _SHAPE[-1]
_NROWS = 64
_S = 48


def _copy_kernel(in_ref, out_ref, *scratch):
    bufs = scratch[:_S]
    rsems = scratch[_S:2 * _S]
    wsems = scratch[2 * _S:3 * _S]

    def row_idx(r):
        return r >> 3, (r >> 1) & 3, r & 1

    def read(i):
        s = i % _S
        b, st, ch = row_idx(i)
        return pltpu.make_async_copy(in_ref.at[b, st, ch], bufs[s], rsems[s])

    def write(i):
        s = i % _S
        b, st, ch = row_idx(i)
        return pltpu.make_async_copy(bufs[s], out_ref.at[b, st, ch], wsems[s])

    for k in range(_S):
        read(k).start()
    for i in range(_NROWS):
        if i >= 1 and i - 1 + _S < _NROWS:
            write(i - 1).wait()
            read(i - 1 + _S).start()
        read(i).wait()
        write(i).start()
    for i in range(_NROWS - _S, _NROWS):
        write(i).wait()


def kernel(stems):
    return pl.pallas_call(
        _copy_kernel,
        out_shape=jax.ShapeDtypeStruct(_SHAPE, jnp.float32),
        in_specs=[pl.BlockSpec(memory_space=pltpu.MemorySpace.HBM)],
        out_specs=pl.BlockSpec(memory_space=pltpu.MemorySpace.HBM),
        compiler_params=pltpu.CompilerParams(vmem_limit_bytes=100 * 1024 * 1024),
        scratch_shapes=(
            [pltpu.VMEM((_L,), jnp.float32)] * _S
            + [pltpu.SemaphoreType.DMA] * (2 * _S)
        ),
    )(stems)
